# baseline (speedup 1.0000x reference)
import functools

import jax
import jax.numpy as jnp
from jax import lax
from jax.experimental import pallas as pl
from jax.experimental.pallas import tpu as pltpu
from jax.experimental.pallas import tpu_sc as plsc

D = 64
ROW_LEN = 50
IDX_PAD = 64
GATHER_W = 56
NUM_WORKERS = 32
K = 8


def _idx_prep(x):
    n = x.shape[0]
    br = 2048

    def body(x_ref, o_ref):
        xb = x_ref[...]
        o_ref[...] = jnp.concatenate(
            [xb, xb[:, ROW_LEN - (IDX_PAD - ROW_LEN):]], axis=1)

    return pl.pallas_call(
        body,
        grid=(n // br,),
        in_specs=[pl.BlockSpec((br, ROW_LEN), lambda i: (i, 0))],
        out_specs=pl.BlockSpec((br, IDX_PAD), lambda i: (i, 0)),
        out_shape=jax.ShapeDtypeStruct((n, IDX_PAD), jnp.int32),
    )(x)


def _post_slice(g):
    n = g.shape[0]
    br = 256

    def body(g_ref, o_ref):
        o_ref[...] = g_ref[:, :ROW_LEN, :]

    return pl.pallas_call(
        body,
        grid=(n // br,),
        in_specs=[pl.BlockSpec((br, GATHER_W, D), lambda i: (i, 0, 0))],
        out_specs=pl.BlockSpec((br, ROW_LEN, D), lambda i: (i, 0, 0)),
        out_shape=jax.ShapeDtypeStruct((n, ROW_LEN, D), jnp.float32),
    )(g)


def _make_sc_kernel(num_rows):
    rows_per_w = num_rows // NUM_WORKERS
    num_chunks = rows_per_w // K
    assert rows_per_w % K == 0 and num_chunks % 2 == 0 and num_chunks >= 6
    mesh = plsc.VectorSubcoreMesh(core_axis_name="c", subcore_axis_name="s")

    @functools.partial(
        pl.kernel,
        out_type=jax.ShapeDtypeStruct((num_rows, GATHER_W, D), jnp.float32),
        mesh=mesh,
        scratch_types=[
            pltpu.VMEM((2, K, IDX_PAD), jnp.int32),
            pltpu.VMEM((2, K, GATHER_W, D), jnp.float32),
            pltpu.SemaphoreType.DMA,
            pltpu.SemaphoreType.DMA,
            pltpu.SemaphoreType.DMA,
            pltpu.SemaphoreType.DMA,
            pltpu.SemaphoreType.DMA,
        ],
        compiler_params=pltpu.CompilerParams(use_tc_tiling_on_sc=False),
    )
    def emb(table_hbm, idx_hbm, out_hbm, idx_v, rows_v, gsem,
            isem0, isem1, osem0, osem1):
        wid = lax.axis_index("s") * 2 + lax.axis_index("c")
        base_row = wid * rows_per_w
        isem = (isem0, isem1)
        osem = (osem0, osem1)

        def idx_start(c, b):
            pltpu.async_copy(
                idx_hbm.at[pl.ds(base_row + c * K, K)],
                idx_v.at[b], isem[b])

        def idx_wait(c, b):
            pltpu.make_async_copy(
                idx_hbm.at[pl.ds(base_row + c * K, K)],
                idx_v.at[b], isem[b]).wait()

        def gather(b):
            copies = [
                pltpu.async_copy(
                    table_hbm.at[idx_v.at[b, j, pl.ds(0, GATHER_W)]],
                    rows_v.at[b, j], gsem)
                for j in range(K)
            ]
            for cp in copies:
                cp.wait()

        def out_start(c, b):
            pltpu.async_copy(
                rows_v.at[b], out_hbm.at[pl.ds(base_row + c * K, K)], osem[b])

        def out_wait(c, b):
            pltpu.make_async_copy(
                rows_v.at[b], out_hbm.at[pl.ds(base_row + c * K, K)],
                osem[b]).wait()

        idx_start(0, 0)
        idx_start(1, 1)
        for b in range(2):
            idx_wait(b, b)
            gather(b)
            out_start(b, b)
            idx_start(b + 2, b)

        @pl.loop(2, num_chunks - 2, step=2)
        def body(c0):
            for b in range(2):
                c = c0 + b
                idx_wait(c, b)
                out_wait(c - 2, b)
                gather(b)
                out_start(c, b)
                idx_start(c + 2, b)

        for b in range(2):
            c = num_chunks - 2 + b
            idx_wait(c, b)
            out_wait(c - 2, b)
            gather(b)
            out_start(c, b)
        for b in range(2):
            out_wait(num_chunks - 2 + b, b)

    return emb


def kernel(x, W_embed):
    b0, _ = x.shape
    idx = _idx_prep(x.astype(jnp.int32))
    gathered = _make_sc_kernel(b0)(W_embed, idx)
    return _post_slice(gathered)


# speedup vs baseline: 1.3446x; 1.3446x over previous
import functools

import jax
import jax.numpy as jnp
from jax import lax
from jax.experimental import pallas as pl
from jax.experimental.pallas import tpu as pltpu
from jax.experimental.pallas import tpu_sc as plsc

D = 64
ROW_LEN = 50
IDX_PAD = 64
GATHER_W = 56
NUM_WORKERS = 32
K = 8


def _idx_prep(x):
    n = x.shape[0]
    br = 2048

    def body(x_ref, o_ref):
        xb = x_ref[...]
        o_ref[...] = jnp.concatenate(
            [xb, xb[:, ROW_LEN - (IDX_PAD - ROW_LEN):]], axis=1)

    return pl.pallas_call(
        body,
        grid=(n // br,),
        in_specs=[pl.BlockSpec((br, ROW_LEN), lambda i: (i, 0))],
        out_specs=pl.BlockSpec((br, IDX_PAD), lambda i: (i, 0)),
        out_shape=jax.ShapeDtypeStruct((n, IDX_PAD), jnp.int32),
    )(x)


def _make_sc_kernel(num_rows):
    rows_per_w = num_rows // NUM_WORKERS
    num_chunks = rows_per_w // K
    assert rows_per_w % K == 0 and num_chunks % 2 == 0 and num_chunks >= 4
    mesh = plsc.VectorSubcoreMesh(core_axis_name="c", subcore_axis_name="s")

    @functools.partial(
        pl.kernel,
        out_type=jax.ShapeDtypeStruct((num_rows, ROW_LEN, D), jnp.float32),
        mesh=mesh,
        scratch_types=[
            pltpu.VMEM((2, K, IDX_PAD), jnp.int32),
            pltpu.VMEM((2, K, GATHER_W, D), jnp.float32),
            pltpu.SemaphoreType.DMA,
            pltpu.SemaphoreType.DMA((2,)),
            pltpu.SemaphoreType.DMA((2,)),
        ],
        compiler_params=pltpu.CompilerParams(use_tc_tiling_on_sc=False),
    )
    def emb(table_hbm, idx_hbm, out_hbm, idx_v, rows_v, gsem, isem, osem):
        wid = lax.axis_index("s") * 2 + lax.axis_index("c")
        base_row = wid * rows_per_w

        def idx_start(c, b):
            pltpu.async_copy(
                idx_hbm.at[pl.ds(base_row + c * K, K)],
                idx_v.at[b], isem.at[b])

        def idx_wait(c, b):
            pltpu.make_async_copy(
                idx_hbm.at[pl.ds(base_row + c * K, K)],
                idx_v.at[b], isem.at[b]).wait()

        def out_start(c, b):
            pltpu.async_copy(
                rows_v.at[b, :, pl.ds(0, ROW_LEN)],
                out_hbm.at[pl.ds(base_row + c * K, K)], osem.at[b])

        def out_wait(c, b):
            pltpu.make_async_copy(
                rows_v.at[b, :, pl.ds(0, ROW_LEN)],
                out_hbm.at[pl.ds(base_row + c * K, K)], osem.at[b]).wait()

        idx_start(0, 0)
        idx_start(1, 1)

        @pl.loop(0, num_chunks)
        def chunk(c):
            b = c % 2
            idx_wait(c, b)

            @pl.when(c >= 2)
            def _():
                out_wait(c - 2, b)

            @pl.loop(0, K)
            def fire(j):
                pltpu.async_copy(
                    table_hbm.at[idx_v.at[b, j, pl.ds(0, GATHER_W)]],
                    rows_v.at[b, j], gsem)

            @pl.loop(0, K)
            def drain(j):
                pltpu.make_async_copy(
                    table_hbm.at[idx_v.at[b, j, pl.ds(0, GATHER_W)]],
                    rows_v.at[b, j], gsem).wait()

            out_start(c, b)

            @pl.when(c + 2 < num_chunks)
            def _():
                idx_start(c + 2, b)

        out_wait(num_chunks - 2, 0)
        out_wait(num_chunks - 1, 1)

    return emb


def kernel(x, W_embed):
    b0, _ = x.shape
    idx = _idx_prep(x.astype(jnp.int32))
    return _make_sc_kernel(b0)(W_embed, idx)
